# Initial kernel scaffold; baseline (speedup 1.0000x reference)
import jax
import jax.numpy as jnp
from jax.experimental import pallas as pl


def kernel(x, W, b):
    raise NotImplementedError("write your pallas kernel here")


# speedup vs baseline: 1.1779x; 1.1779x over previous
import functools

import jax
import jax.numpy as jnp
from jax import lax
from jax.experimental import pallas as pl
from jax.experimental.pallas import tpu as pltpu
from jax.experimental.pallas import tpu_sc as plsc


def _take(v, idx):
    return lax.gather(
        v,
        idx[:, None],
        lax.GatherDimensionNumbers(
            offset_dims=(), collapsed_slice_dims=(0,), start_index_map=(0,)),
        slice_sizes=(1,),
        mode=lax.GatherScatterMode.PROMISE_IN_BOUNDS,
    )


def _body(pack_hbm, w_hbm, out_hbm, pack_v, w_v, out_v):
    c = lax.axis_index("c")
    s = lax.axis_index("s")

    @pl.when((c == 0) & (s == 0))
    def _():
        pltpu.sync_copy(pack_hbm, pack_v)
        pltpu.sync_copy(w_hbm, w_v)

        xv = pack_v[...]
        i = lax.iota(jnp.int32, 16)

        idx1 = jnp.where(i < 2, i + 4,
                         jnp.where(i < 4, i - 2,
                                   jnp.where(i < 6, i + 2, 12)))
        val1 = jnp.where(i < 4, 1.0, jnp.where(i < 6, 3.0, 0.0)).astype(jnp.float32)
        idx2 = jnp.where(i < 2, i + 6, 12)
        val2 = jnp.where(i < 2, 2.0, 0.0).astype(jnp.float32)
        flat = _take(xv, idx1) * val1 + _take(xv, idx2) * val2

        wv0 = w_v[0]
        wv1 = w_v[1]
        fb0 = _take(flat, i >> 2)
        fb1 = _take(flat, (i >> 2) + 4)
        prod = wv0 * fb0 + wv1 * fb1

        r1 = prod + _take(prod, (i + 8) & 15)
        r2 = r1 + _take(r1, (i + 4) & 15)

        bv = _take(xv, (i & 3) + 8)
        out_v[...] = r2 + bv
        pltpu.sync_copy(out_v, out_hbm)


@functools.partial(
    pl.kernel,
    out_type=jax.ShapeDtypeStruct((16,), jnp.float32),
    mesh=plsc.VectorSubcoreMesh(core_axis_name="c", subcore_axis_name="s"),
    scratch_types=[
        pltpu.VMEM((16,), jnp.float32),
        pltpu.VMEM((2, 16), jnp.float32),
        pltpu.VMEM((16,), jnp.float32),
    ],
)
def _sc_kernel(pack_hbm, w_hbm, out_hbm, pack_v, w_v, out_v):
    _body(pack_hbm, w_hbm, out_hbm, pack_v, w_v, out_v)


def kernel(x, W, b):
    pack = jnp.concatenate([x.reshape(-1), b, jnp.zeros((4,), jnp.float32)])
    wt = W.T.reshape(-1)
    wpack = jnp.concatenate([wt, jnp.zeros((8,), jnp.float32)]).reshape(2, 16)
    out16 = _sc_kernel(pack, wpack)
    return out16[:4]


# speedup vs baseline: 1.2440x; 1.0560x over previous
import functools

import jax
import jax.numpy as jnp
from jax import lax
from jax.experimental import pallas as pl
from jax.experimental.pallas import tpu as pltpu
from jax.experimental.pallas import tpu_sc as plsc


def _take(v, idx):
    return lax.gather(
        v,
        idx[:, None],
        lax.GatherDimensionNumbers(
            offset_dims=(), collapsed_slice_dims=(0,), start_index_map=(0,)),
        slice_sizes=(1,),
        mode=lax.GatherScatterMode.PROMISE_IN_BOUNDS,
    )


def _body(x_hbm, w_hbm, b_hbm, out_hbm, x_v, w_v, b_v, out_v, sem):
    c = lax.axis_index("c")
    s = lax.axis_index("s")

    @pl.when((c == 0) & (s == 0))
    def _():
        cp_x = pltpu.async_copy(x_hbm, x_v.at[pl.ds(0, 8)], sem)
        cp_w = pltpu.async_copy(w_hbm, w_v.at[pl.ds(0, 24)], sem)
        cp_b = pltpu.async_copy(b_hbm, b_v.at[pl.ds(0, 4)], sem)
        cp_x.wait()
        cp_w.wait()
        cp_b.wait()

        i = lax.iota(jnp.int32, 16)
        two = i < 2

        x_r = x_v[...]
        w0 = w_v[pl.ds(0, 16)]
        w1 = w_v[pl.ds(16, 16)]
        b_r = b_v[...]

        idx1 = jnp.where(two, i + 4,
                         jnp.where(i < 4, i - 2, jnp.where(i < 6, i + 2, 0)))
        val1 = jnp.where(i < 4, 1.0, jnp.where(i < 6, 3.0, 0.0)).astype(jnp.float32)
        idx2 = jnp.where(two, i + 6, 0)
        val2 = jnp.where(two, 2.0, 0.0).astype(jnp.float32)
        flat = _take(x_r, idx1) * val1 + _take(x_r, idx2) * val2

        j4 = i & 3
        k0 = i >> 2
        k1 = jnp.where(i < 8, k0 + 4, 0)
        t0 = j4 * 6 + k0
        t1 = j4 * 6 + k1
        wv0 = jnp.where(t0 < 16, _take(w0, t0 & 15), _take(w1, t0 & 15))
        wv1 = jnp.where(t1 < 16, _take(w0, t1 & 15), _take(w1, t1 & 15))
        fb0 = _take(flat, k0)
        fb1 = _take(flat, jnp.where(i < 8, k0 + 4, 6))
        prod = wv0 * fb0 + wv1 * fb1

        s1 = prod + _take(prod, (i + 8) & 15)
        s2 = s1 + _take(s1, (i + 4) & 15)

        out_v[...] = s2 + _take(b_r, j4)
        pltpu.sync_copy(out_v.at[pl.ds(0, 4)], out_hbm)


@functools.partial(
    pl.kernel,
    out_type=jax.ShapeDtypeStruct((4,), jnp.float32),
    mesh=plsc.VectorSubcoreMesh(core_axis_name="c", subcore_axis_name="s"),
    scratch_types=[
        pltpu.VMEM((16,), jnp.float32),
        pltpu.VMEM((32,), jnp.float32),
        pltpu.VMEM((16,), jnp.float32),
        pltpu.VMEM((16,), jnp.float32),
        pltpu.SemaphoreType.DMA,
    ],
)
def _sc_kernel(x_hbm, w_hbm, b_hbm, out_hbm, x_v, w_v, b_v, out_v, sem):
    _body(x_hbm, w_hbm, b_hbm, out_hbm, x_v, w_v, b_v, out_v, sem)


def kernel(x, W, b):
    return _sc_kernel(x.reshape(8), W.reshape(24), b)


# speedup vs baseline: 1.3524x; 1.0872x over previous
import functools

import jax
import jax.numpy as jnp
from jax import lax
from jax.experimental import pallas as pl
from jax.experimental.pallas import tpu as pltpu
from jax.experimental.pallas import tpu_sc as plsc


def _take(v, idx):
    return lax.gather(
        v,
        idx[:, None],
        lax.GatherDimensionNumbers(
            offset_dims=(), collapsed_slice_dims=(0,), start_index_map=(0,)),
        slice_sizes=(1,),
        mode=lax.GatherScatterMode.PROMISE_IN_BOUNDS,
    )


def _body(x_hbm, w_hbm, b_hbm, out_hbm, x_v, w_v, b_v, out_v, sem):
    c = lax.axis_index("c")
    s = lax.axis_index("s")

    @pl.when((c == 0) & (s == 0))
    def _():
        cp_x = pltpu.async_copy(x_hbm, x_v.at[pl.ds(0, 8)], sem)
        cp_w = pltpu.async_copy(w_hbm, w_v.at[pl.ds(0, 24)], sem)
        cp_b = pltpu.async_copy(b_hbm, b_v.at[pl.ds(0, 4)], sem)
        cp_x.wait()
        cp_w.wait()
        cp_b.wait()

        i = lax.iota(jnp.int32, 16)
        two = i < 2

        x_r = x_v[...]
        w0 = w_v[pl.ds(0, 16)]
        w1 = w_v[pl.ds(16, 16)]
        b_r = b_v[...]

        idx1 = jnp.where(two, i + 4,
                         jnp.where(i < 4, i - 2, jnp.where(i < 6, i + 2, 0)))
        val1 = jnp.where(i < 4, 1.0, jnp.where(i < 6, 3.0, 0.0)).astype(jnp.float32)
        idx2 = jnp.where(two, i + 6, 0)
        val2 = jnp.where(two, 2.0, 0.0).astype(jnp.float32)
        flat = _take(x_r, idx1) * val1 + _take(x_r, idx2) * val2

        j4 = i & 3
        k0 = i >> 2
        k1 = jnp.where(i < 8, k0 + 4, 0)
        t0 = j4 * 6 + k0
        t1 = j4 * 6 + k1
        wv0 = jnp.where(t0 < 16, _take(w0, t0 & 15), _take(w1, t0 & 15))
        wv1 = jnp.where(t1 < 16, _take(w0, t1 & 15), _take(w1, t1 & 15))
        fb0 = _take(flat, k0)
        fb1 = _take(flat, jnp.where(i < 8, k0 + 4, 6))
        prod = wv0 * fb0 + wv1 * fb1

        s1 = prod + _take(prod, (i + 8) & 15)
        s2 = s1 + _take(s1, (i + 4) & 15)

        out_v[...] = s2 + _take(b_r, j4)
        pltpu.sync_copy(out_v.at[pl.ds(0, 4)], out_hbm)


@functools.partial(
    pl.kernel,
    out_type=jax.ShapeDtypeStruct((4,), jnp.float32),
    mesh=plsc.VectorSubcoreMesh(
        core_axis_name="c", subcore_axis_name="s", num_cores=1),
    scratch_types=[
        pltpu.VMEM((16,), jnp.float32),
        pltpu.VMEM((32,), jnp.float32),
        pltpu.VMEM((16,), jnp.float32),
        pltpu.VMEM((16,), jnp.float32),
        pltpu.SemaphoreType.DMA,
    ],
)
def _sc_kernel(x_hbm, w_hbm, b_hbm, out_hbm, x_v, w_v, b_v, out_v, sem):
    _body(x_hbm, w_hbm, b_hbm, out_hbm, x_v, w_v, b_v, out_v, sem)


def kernel(x, W, b):
    return _sc_kernel(x.reshape(8), W.reshape(24), b)


# speedup vs baseline: 1.4628x; 1.0816x over previous
import functools

import jax
import jax.numpy as jnp
from jax import lax
from jax.experimental import pallas as pl
from jax.experimental.pallas import tpu as pltpu
from jax.experimental.pallas import tpu_sc as plsc


def _body(x_hbm, w_hbm, b_hbm, out_hbm, x_s, w_s, b_s, out_s, sem):
    cp_x = pltpu.async_copy(x_hbm, x_s, sem)
    cp_w = pltpu.async_copy(w_hbm, w_s, sem)
    cp_b = pltpu.async_copy(b_hbm, b_s, sem)
    cp_x.wait()
    cp_w.wait()
    cp_b.wait()

    flat = (
        x_s[4] + 2.0 * x_s[6],
        x_s[5] + 2.0 * x_s[7],
        x_s[0],
        x_s[1],
        3.0 * x_s[6],
        3.0 * x_s[7],
    )
    for j in range(4):
        acc = b_s[j]
        for k in range(6):
            acc = acc + w_s[6 * j + k] * flat[k]
        out_s[j] = acc
    pltpu.sync_copy(out_s, out_hbm)


@functools.partial(
    pl.kernel,
    out_type=jax.ShapeDtypeStruct((4,), jnp.float32),
    mesh=plsc.ScalarSubcoreMesh(axis_name="c", num_cores=1),
    scratch_types=[
        pltpu.SMEM((8,), jnp.float32),
        pltpu.SMEM((24,), jnp.float32),
        pltpu.SMEM((4,), jnp.float32),
        pltpu.SMEM((4,), jnp.float32),
        pltpu.SemaphoreType.DMA,
    ],
)
def _sc_kernel(x_hbm, w_hbm, b_hbm, out_hbm, x_s, w_s, b_s, out_s, sem):
    _body(x_hbm, w_hbm, b_hbm, out_hbm, x_s, w_s, b_s, out_s, sem)


def kernel(x, W, b):
    return _sc_kernel(x.reshape(8), W.reshape(24), b)
